# baseline (speedup 1.0000x reference)
import functools

import jax
import jax.numpy as jnp
import numpy as np
from jax.experimental import pallas as pl
from jax.experimental.pallas import tpu as pltpu
from jax.sharding import Mesh, PartitionSpec as P

_B, _T, _D, _H = 8, 2048, 256, 256
_CHUNK = 256
_NCH = _T // _CHUNK


def _layer_phase_kernel(in_ref, wih_ref, whh_ref, bih_ref, bhh_ref, h_ref,
                        y_ref, hout_ref, gi_ref):
    gi_ref[...] = (
        jnp.dot(in_ref[...], wih_ref[...], preferred_element_type=jnp.float32)
        + bih_ref[...]
    )

    def step(i, h):
        hb16 = h.astype(jnp.bfloat16)
        gi = gi_ref[pl.ds(i * _B, _B)]
        gh_r = jnp.dot(hb16, whh_ref[:, :_H],
                       preferred_element_type=jnp.float32) + bhh_ref[:, :_H]
        gh_z = jnp.dot(hb16, whh_ref[:, _H:2 * _H],
                       preferred_element_type=jnp.float32) + bhh_ref[:, _H:2 * _H]
        gh_n = jnp.dot(hb16, whh_ref[:, 2 * _H:],
                       preferred_element_type=jnp.float32) + bhh_ref[:, 2 * _H:]
        r = jax.nn.sigmoid(gi[:, :_H] + gh_r)
        z = jax.nn.sigmoid(gi[:, _H:2 * _H] + gh_z)
        n = jnp.tanh(gi[:, 2 * _H:] + r * gh_n)
        h_new = (1.0 - z) * n + z * h
        y_ref[pl.ds(i * _B, _B)] = h_new
        return h_new

    hout_ref[...] = jax.lax.fori_loop(0, _CHUNK, step, h_ref[...], unroll=16)


def _layer_phase(in_buf, wih, whh, bih, bhh, h):
    return pl.pallas_call(
        _layer_phase_kernel,
        out_shape=(
            jax.ShapeDtypeStruct((_CHUNK * _B, _H), jnp.float32),
            jax.ShapeDtypeStruct((_B, _H), jnp.float32),
        ),
        scratch_shapes=[
            pltpu.VMEM((_CHUNK * _B, 3 * _H), jnp.float32),
        ],
    )(in_buf, wih, whh, bih, bhh, h)


def _spmd(xt, wih, whh, bih, bhh):
    d = jax.lax.axis_index('c')
    wih, whh, bih, bhh = wih[0], whh[0], bih[0], bhh[0]
    h = jnp.zeros((_B, _H), jnp.float32)
    recv = jnp.zeros((_CHUNK * _B, _H), jnp.bfloat16)
    outs = []
    for c in range(_NCH + 1):
        cc = min(c, _NCH - 1)
        x_chunk = xt[cc * _CHUNK * _B:(cc + 1) * _CHUNK * _B]
        in_buf = jnp.where(d == 0, x_chunk, recv)
        y_chunk, h = _layer_phase(in_buf, wih, whh, bih, bhh, h)
        if c == 0:
            h = jnp.where(d == 0, h, jnp.zeros_like(h))
        else:
            outs.append(y_chunk)
        if c < _NCH:
            recv = jax.lax.ppermute(y_chunk.astype(jnp.bfloat16), 'c',
                                    [(0, 1)])
    return jnp.concatenate(outs, axis=0)[None]


@jax.jit
def kernel(x, w_ih_l0, w_hh_l0, b_ih_l0, b_hh_l0,
           w_ih_l1, w_hh_l1, b_ih_l1, b_hh_l1):
    xt = jnp.swapaxes(x, 0, 1).reshape(_T * _B, _D).astype(jnp.bfloat16)
    wih = jnp.stack([w_ih_l0.T, w_ih_l1.T]).astype(jnp.bfloat16)
    whh = jnp.stack([w_hh_l0.T, w_hh_l1.T]).astype(jnp.bfloat16)
    bih = jnp.stack([b_ih_l0[None], b_ih_l1[None]])
    bhh = jnp.stack([b_hh_l0[None], b_hh_l1[None]])

    mesh = Mesh(np.array(jax.devices()[:2]), ('c',))
    y2 = jax.shard_map(
        _spmd, mesh=mesh, check_vma=False,
        in_specs=(P(), P('c'), P('c'), P('c'), P('c')),
        out_specs=P('c'),
    )(xt, wih, whh, bih, bhh)
    y = y2[1]
    return jnp.swapaxes(y.reshape(_T, _B, _H), 0, 1)


# speedup vs baseline: 1.4124x; 1.4124x over previous
import jax
import jax.numpy as jnp
from jax.experimental import pallas as pl
from jax.experimental.pallas import tpu as pltpu

_B, _T, _D, _H = 8, 2048, 256, 256
_CHUNK = 256
_NCH = _T // _CHUNK


def _gru2_kernel(x_ref, wih0_ref, whh0_ref, bih0_ref, bhh0_ref,
                 wih1_ref, whh1_ref, bih1_ref, bhh1_ref,
                 y_ref, h0_ref, h1_ref, gi0_ref, gi1_ref, y0_ref):
    c = pl.program_id(0)

    @pl.when(c == 0)
    def _init0():
        h0_ref[...] = jnp.zeros_like(h0_ref)

    @pl.when(c <= 1)
    def _init1():
        h1_ref[...] = jnp.zeros_like(h1_ref)

    gi0_ref[...] = (
        jnp.dot(x_ref[...], wih0_ref[...], preferred_element_type=jnp.float32)
        + bih0_ref[...]
    )

    def gates(g_i, g_h, h):
        r = jax.nn.sigmoid(g_i[:, :_H] + g_h[:, :_H])
        z = jax.nn.sigmoid(g_i[:, _H:2 * _H] + g_h[:, _H:2 * _H])
        n = jnp.tanh(g_i[:, 2 * _H:] + r * g_h[:, 2 * _H:])
        return (1.0 - z) * n + z * h

    def body(i, carry):
        h0, h1 = carry
        gi0 = gi0_ref[pl.ds(i * _B, _B)]
        gi1 = gi1_ref[pl.ds(i * _B, _B)]
        h0b = h0.astype(jnp.bfloat16)
        h1b = h1.astype(jnp.bfloat16)
        f32 = jnp.float32
        gh0_r = jnp.dot(h0b, whh0_ref[:, :_H], preferred_element_type=f32)
        gh1_r = jnp.dot(h1b, whh1_ref[:, :_H], preferred_element_type=f32)
        gh0_z = jnp.dot(h0b, whh0_ref[:, _H:2 * _H], preferred_element_type=f32)
        gh1_z = jnp.dot(h1b, whh1_ref[:, _H:2 * _H], preferred_element_type=f32)
        gh0_n = jnp.dot(h0b, whh0_ref[:, 2 * _H:], preferred_element_type=f32)
        gh1_n = jnp.dot(h1b, whh1_ref[:, 2 * _H:], preferred_element_type=f32)
        r0 = jax.nn.sigmoid(gi0[:, :_H] + (gh0_r + bhh0_ref[:, :_H]))
        r1 = jax.nn.sigmoid(gi1[:, :_H] + (gh1_r + bhh1_ref[:, :_H]))
        z0 = jax.nn.sigmoid(gi0[:, _H:2 * _H] + (gh0_z + bhh0_ref[:, _H:2 * _H]))
        z1 = jax.nn.sigmoid(gi1[:, _H:2 * _H] + (gh1_z + bhh1_ref[:, _H:2 * _H]))
        n0 = jnp.tanh(gi0[:, 2 * _H:] + r0 * (gh0_n + bhh0_ref[:, 2 * _H:]))
        n1 = jnp.tanh(gi1[:, 2 * _H:] + r1 * (gh1_n + bhh1_ref[:, 2 * _H:]))
        h0_next = (1.0 - z0) * n0 + z0 * h0
        h1_next = (1.0 - z1) * n1 + z1 * h1
        y0_ref[pl.ds(i * _B, _B)] = h0_next
        y_ref[pl.ds(i * _B, _B)] = h1_next
        return h0_next, h1_next

    h0, h1 = jax.lax.fori_loop(0, _CHUNK, body, (h0_ref[...], h1_ref[...]),
                               unroll=16)
    h0_ref[...] = h0
    h1_ref[...] = h1

    gi1_ref[...] = (
        jnp.dot(y0_ref[...].astype(jnp.bfloat16), wih1_ref[...],
                preferred_element_type=jnp.float32)
        + bih1_ref[...]
    )


@jax.jit
def kernel(x, w_ih_l0, w_hh_l0, b_ih_l0, b_hh_l0,
           w_ih_l1, w_hh_l1, b_ih_l1, b_hh_l1):
    xt = jnp.swapaxes(x, 0, 1).reshape(_T * _B, _D).astype(jnp.bfloat16)

    full = lambda shape: pl.BlockSpec(shape, lambda c: (0,) * len(shape))
    y = pl.pallas_call(
        _gru2_kernel,
        grid=(_NCH + 1,),
        in_specs=[
            pl.BlockSpec((_CHUNK * _B, _D),
                         lambda c: (jnp.minimum(c, _NCH - 1), 0)),
            full((_D, 3 * _H)),
            full((_H, 3 * _H)),
            full((1, 3 * _H)),
            full((1, 3 * _H)),
            full((_H, 3 * _H)),
            full((_H, 3 * _H)),
            full((1, 3 * _H)),
            full((1, 3 * _H)),
        ],
        out_specs=pl.BlockSpec((_CHUNK * _B, _H),
                               lambda c: (jnp.maximum(c - 1, 0), 0)),
        out_shape=jax.ShapeDtypeStruct((_T * _B, _H), jnp.float32),
        scratch_shapes=[
            pltpu.VMEM((_B, _H), jnp.float32),
            pltpu.VMEM((_B, _H), jnp.float32),
            pltpu.VMEM((_CHUNK * _B, 3 * _H), jnp.float32),
            pltpu.VMEM((_CHUNK * _B, 3 * _H), jnp.float32),
            pltpu.VMEM((_CHUNK * _B, _H), jnp.float32),
        ],
        compiler_params=pltpu.CompilerParams(
            dimension_semantics=("arbitrary",),
        ),
    )(
        xt,
        w_ih_l0.T.astype(jnp.bfloat16), w_hh_l0.T.astype(jnp.bfloat16),
        b_ih_l0[None], b_hh_l0[None],
        w_ih_l1.T.astype(jnp.bfloat16), w_hh_l1.T.astype(jnp.bfloat16),
        b_ih_l1[None], b_hh_l1[None],
    )
    return jnp.swapaxes(y.reshape(_T, _B, _H), 0, 1)


# speedup vs baseline: 1.6734x; 1.1848x over previous
import jax
import jax.numpy as jnp
from jax.experimental import pallas as pl
from jax.experimental.pallas import tpu as pltpu

_B, _T, _D, _H = 8, 2048, 256, 256
_CHUNK = 256
_NCH = _T // _CHUNK


def _gru2_kernel(x_ref, wih0_ref, whh0_ref, bih0_ref, bhh0_ref,
                 wih1_ref, whh1_ref, bih1_ref, bhh1_ref,
                 y_ref, h0_ref, h1_ref, gi0_ref, gi1_ref, y0_ref):
    c = pl.program_id(0)

    @pl.when(c == 0)
    def _init0():
        h0_ref[...] = jnp.zeros_like(h0_ref)

    @pl.when(c <= 1)
    def _init1():
        h1_ref[...] = jnp.zeros_like(h1_ref)

    gi0_ref[...] = (
        jnp.dot(x_ref[...], wih0_ref[...], preferred_element_type=jnp.float32)
        + bih0_ref[...]
    )

    def gates(g_i, g_h, h):
        r = jax.nn.sigmoid(g_i[:, :_H] + g_h[:, :_H])
        z = jax.nn.sigmoid(g_i[:, _H:2 * _H] + g_h[:, _H:2 * _H])
        n = jnp.tanh(g_i[:, 2 * _H:] + r * g_h[:, 2 * _H:])
        return (1.0 - z) * n + z * h

    def layer_step(h, w_ref, b_ref, gi):
        hb16 = h.astype(jnp.bfloat16)
        gh_r = jnp.dot(hb16, w_ref[:, :_H],
                       preferred_element_type=jnp.float32) + b_ref[:, :_H]
        gh_z = jnp.dot(hb16, w_ref[:, _H:2 * _H],
                       preferred_element_type=jnp.float32) + b_ref[:, _H:2 * _H]
        gh_n = jnp.dot(hb16, w_ref[:, 2 * _H:],
                       preferred_element_type=jnp.float32) + b_ref[:, 2 * _H:]
        r = jax.nn.sigmoid(gi[:, :_H] + gh_r)
        z = jax.nn.sigmoid(gi[:, _H:2 * _H] + gh_z)
        n = jnp.tanh(gi[:, 2 * _H:] + r * gh_n)
        return (1.0 - z) * n + z * h

    def body(i, carry):
        h0, h1 = carry
        h0_next = layer_step(h0, whh0_ref, bhh0_ref, gi0_ref[pl.ds(i * _B, _B)])
        h1_next = layer_step(h1, whh1_ref, bhh1_ref, gi1_ref[pl.ds(i * _B, _B)])
        y0_ref[pl.ds(i * _B, _B)] = h0_next
        y_ref[pl.ds(i * _B, _B)] = h1_next
        return h0_next, h1_next

    h0, h1 = jax.lax.fori_loop(0, _CHUNK, body, (h0_ref[...], h1_ref[...]),
                               unroll=32)
    h0_ref[...] = h0
    h1_ref[...] = h1

    gi1_ref[...] = (
        jnp.dot(y0_ref[...].astype(jnp.bfloat16), wih1_ref[...],
                preferred_element_type=jnp.float32)
        + bih1_ref[...]
    )


@jax.jit
def kernel(x, w_ih_l0, w_hh_l0, b_ih_l0, b_hh_l0,
           w_ih_l1, w_hh_l1, b_ih_l1, b_hh_l1):
    xt = jnp.swapaxes(x, 0, 1).reshape(_T * _B, _D).astype(jnp.bfloat16)

    full = lambda shape: pl.BlockSpec(shape, lambda c: (0,) * len(shape))
    y = pl.pallas_call(
        _gru2_kernel,
        grid=(_NCH + 1,),
        in_specs=[
            pl.BlockSpec((_CHUNK * _B, _D),
                         lambda c: (jnp.minimum(c, _NCH - 1), 0)),
            full((_D, 3 * _H)),
            full((_H, 3 * _H)),
            full((1, 3 * _H)),
            full((1, 3 * _H)),
            full((_H, 3 * _H)),
            full((_H, 3 * _H)),
            full((1, 3 * _H)),
            full((1, 3 * _H)),
        ],
        out_specs=pl.BlockSpec((_CHUNK * _B, _H),
                               lambda c: (jnp.maximum(c - 1, 0), 0)),
        out_shape=jax.ShapeDtypeStruct((_T * _B, _H), jnp.float32),
        scratch_shapes=[
            pltpu.VMEM((_B, _H), jnp.float32),
            pltpu.VMEM((_B, _H), jnp.float32),
            pltpu.VMEM((_CHUNK * _B, 3 * _H), jnp.float32),
            pltpu.VMEM((_CHUNK * _B, 3 * _H), jnp.float32),
            pltpu.VMEM((_CHUNK * _B, _H), jnp.float32),
        ],
        compiler_params=pltpu.CompilerParams(
            dimension_semantics=("arbitrary",),
        ),
    )(
        xt,
        w_ih_l0.T.astype(jnp.bfloat16), w_hh_l0.T.astype(jnp.bfloat16),
        b_ih_l0[None], b_hh_l0[None],
        w_ih_l1.T.astype(jnp.bfloat16), w_hh_l1.T.astype(jnp.bfloat16),
        b_ih_l1[None], b_hh_l1[None],
    )
    return jnp.swapaxes(y.reshape(_T, _B, _H), 0, 1)


# speedup vs baseline: 1.6818x; 1.0050x over previous
import jax
import jax.numpy as jnp
from jax.experimental import pallas as pl
from jax.experimental.pallas import tpu as pltpu

_B, _T, _D, _H = 8, 2048, 256, 256
_CHUNK = 256
_NCH = _T // _CHUNK


def _gru2_kernel(x_ref, wih0_ref, whh0_ref, bih0_ref, bhh0_ref,
                 wih1_ref, whh1_ref, bih1_ref, bhh1_ref,
                 y_ref, h0_ref, h1_ref, gi0_ref, gi1_ref, y0_ref):
    c = pl.program_id(0)

    @pl.when(c == 0)
    def _init0():
        h0_ref[...] = jnp.zeros_like(h0_ref)

    @pl.when(c <= 1)
    def _init1():
        h1_ref[...] = jnp.zeros_like(h1_ref)

    gi0_ref[...] = (
        jnp.dot(x_ref[...], wih0_ref[...], preferred_element_type=jnp.float32)
        + bih0_ref[...]
    )

    def gates(g_i, g_h, h):
        r = jax.nn.sigmoid(g_i[:, :_H] + g_h[:, :_H])
        z = jax.nn.sigmoid(g_i[:, _H:2 * _H] + g_h[:, _H:2 * _H])
        n = jnp.tanh(g_i[:, 2 * _H:] + r * g_h[:, 2 * _H:])
        return (1.0 - z) * n + z * h

    def layer_step(h, w_ref, b_ref, gi):
        hb16 = h.astype(jnp.bfloat16)
        gh_r = jnp.dot(hb16, w_ref[:, :_H],
                       preferred_element_type=jnp.float32) + b_ref[:, :_H]
        gh_z = jnp.dot(hb16, w_ref[:, _H:2 * _H],
                       preferred_element_type=jnp.float32) + b_ref[:, _H:2 * _H]
        gh_n = jnp.dot(hb16, w_ref[:, 2 * _H:],
                       preferred_element_type=jnp.float32) + b_ref[:, 2 * _H:]
        r = jax.nn.sigmoid(gi[:, :_H] + gh_r)
        z = jax.nn.sigmoid(gi[:, _H:2 * _H] + gh_z)
        n = jnp.tanh(gi[:, 2 * _H:] + r * gh_n)
        return (1.0 - z) * n + z * h

    def body(i, carry):
        h0, h1 = carry
        h0_next = layer_step(h0, whh0_ref, bhh0_ref, gi0_ref[pl.ds(i * _B, _B)])
        h1_next = layer_step(h1, whh1_ref, bhh1_ref, gi1_ref[pl.ds(i * _B, _B)])
        y0_ref[pl.ds(i * _B, _B)] = h0_next
        y_ref[pl.ds(i * _B, _B)] = h1_next
        return h0_next, h1_next

    h0, h1 = jax.lax.fori_loop(0, _CHUNK, body, (h0_ref[...], h1_ref[...]),
                               unroll=64)
    h0_ref[...] = h0
    h1_ref[...] = h1

    gi1_ref[...] = (
        jnp.dot(y0_ref[...].astype(jnp.bfloat16), wih1_ref[...],
                preferred_element_type=jnp.float32)
        + bih1_ref[...]
    )


@jax.jit
def kernel(x, w_ih_l0, w_hh_l0, b_ih_l0, b_hh_l0,
           w_ih_l1, w_hh_l1, b_ih_l1, b_hh_l1):
    xt = jnp.swapaxes(x, 0, 1).reshape(_T * _B, _D).astype(jnp.bfloat16)

    full = lambda shape: pl.BlockSpec(shape, lambda c: (0,) * len(shape))
    y = pl.pallas_call(
        _gru2_kernel,
        grid=(_NCH + 1,),
        in_specs=[
            pl.BlockSpec((_CHUNK * _B, _D),
                         lambda c: (jnp.minimum(c, _NCH - 1), 0)),
            full((_D, 3 * _H)),
            full((_H, 3 * _H)),
            full((1, 3 * _H)),
            full((1, 3 * _H)),
            full((_H, 3 * _H)),
            full((_H, 3 * _H)),
            full((1, 3 * _H)),
            full((1, 3 * _H)),
        ],
        out_specs=pl.BlockSpec((_CHUNK * _B, _H),
                               lambda c: (jnp.maximum(c - 1, 0), 0)),
        out_shape=jax.ShapeDtypeStruct((_T * _B, _H), jnp.float32),
        scratch_shapes=[
            pltpu.VMEM((_B, _H), jnp.float32),
            pltpu.VMEM((_B, _H), jnp.float32),
            pltpu.VMEM((_CHUNK * _B, 3 * _H), jnp.float32),
            pltpu.VMEM((_CHUNK * _B, 3 * _H), jnp.float32),
            pltpu.VMEM((_CHUNK * _B, _H), jnp.float32),
        ],
        compiler_params=pltpu.CompilerParams(
            dimension_semantics=("arbitrary",),
        ),
    )(
        xt,
        w_ih_l0.T.astype(jnp.bfloat16), w_hh_l0.T.astype(jnp.bfloat16),
        b_ih_l0[None], b_hh_l0[None],
        w_ih_l1.T.astype(jnp.bfloat16), w_hh_l1.T.astype(jnp.bfloat16),
        b_ih_l1[None], b_hh_l1[None],
    )
    return jnp.swapaxes(y.reshape(_T, _B, _H), 0, 1)


# speedup vs baseline: 1.7519x; 1.0417x over previous
import jax
import jax.numpy as jnp
from jax.experimental import pallas as pl
from jax.experimental.pallas import tpu as pltpu

_B, _T, _D, _H = 8, 2048, 256, 256
_CHUNK = 128
_NCH = _T // _CHUNK


def _gru2_kernel(x_ref, wih0_ref, whh0_ref, bih0_ref, bhh0_ref,
                 wih1_ref, whh1_ref, bih1_ref, bhh1_ref,
                 y_ref, h0_ref, h1_ref, gi0_ref, gi1_ref, y0_ref):
    c = pl.program_id(0)

    @pl.when(c == 0)
    def _init0():
        h0_ref[...] = jnp.zeros_like(h0_ref)

    @pl.when(c <= 1)
    def _init1():
        h1_ref[...] = jnp.zeros_like(h1_ref)

    gi0_ref[...] = (
        jnp.dot(x_ref[...], wih0_ref[...], preferred_element_type=jnp.float32)
        + bih0_ref[...]
    )

    def gates(g_i, g_h, h):
        r = jax.nn.sigmoid(g_i[:, :_H] + g_h[:, :_H])
        z = jax.nn.sigmoid(g_i[:, _H:2 * _H] + g_h[:, _H:2 * _H])
        n = jnp.tanh(g_i[:, 2 * _H:] + r * g_h[:, 2 * _H:])
        return (1.0 - z) * n + z * h

    def layer_step(h, w_ref, b_ref, gi):
        hb16 = h.astype(jnp.bfloat16)
        gh_r = jnp.dot(hb16, w_ref[:, :_H],
                       preferred_element_type=jnp.float32) + b_ref[:, :_H]
        gh_z = jnp.dot(hb16, w_ref[:, _H:2 * _H],
                       preferred_element_type=jnp.float32) + b_ref[:, _H:2 * _H]
        gh_n = jnp.dot(hb16, w_ref[:, 2 * _H:],
                       preferred_element_type=jnp.float32) + b_ref[:, 2 * _H:]
        r = jax.nn.sigmoid(gi[:, :_H] + gh_r)
        z = jax.nn.sigmoid(gi[:, _H:2 * _H] + gh_z)
        n = jnp.tanh(gi[:, 2 * _H:] + r * gh_n)
        return (1.0 - z) * n + z * h

    def body(i, carry):
        h0, h1 = carry
        h0_next = layer_step(h0, whh0_ref, bhh0_ref, gi0_ref[pl.ds(i * _B, _B)])
        h1_next = layer_step(h1, whh1_ref, bhh1_ref, gi1_ref[pl.ds(i * _B, _B)])
        y0_ref[pl.ds(i * _B, _B)] = h0_next
        y_ref[pl.ds(i * _B, _B)] = h1_next
        return h0_next, h1_next

    h0, h1 = jax.lax.fori_loop(0, _CHUNK, body, (h0_ref[...], h1_ref[...]),
                               unroll=64)
    h0_ref[...] = h0
    h1_ref[...] = h1

    gi1_ref[...] = (
        jnp.dot(y0_ref[...].astype(jnp.bfloat16), wih1_ref[...],
                preferred_element_type=jnp.float32)
        + bih1_ref[...]
    )


@jax.jit
def kernel(x, w_ih_l0, w_hh_l0, b_ih_l0, b_hh_l0,
           w_ih_l1, w_hh_l1, b_ih_l1, b_hh_l1):
    xt = jnp.swapaxes(x, 0, 1).reshape(_T * _B, _D).astype(jnp.bfloat16)

    full = lambda shape: pl.BlockSpec(shape, lambda c: (0,) * len(shape))
    y = pl.pallas_call(
        _gru2_kernel,
        grid=(_NCH + 1,),
        in_specs=[
            pl.BlockSpec((_CHUNK * _B, _D),
                         lambda c: (jnp.minimum(c, _NCH - 1), 0)),
            full((_D, 3 * _H)),
            full((_H, 3 * _H)),
            full((1, 3 * _H)),
            full((1, 3 * _H)),
            full((_H, 3 * _H)),
            full((_H, 3 * _H)),
            full((1, 3 * _H)),
            full((1, 3 * _H)),
        ],
        out_specs=pl.BlockSpec((_CHUNK * _B, _H),
                               lambda c: (jnp.maximum(c - 1, 0), 0)),
        out_shape=jax.ShapeDtypeStruct((_T * _B, _H), jnp.float32),
        scratch_shapes=[
            pltpu.VMEM((_B, _H), jnp.float32),
            pltpu.VMEM((_B, _H), jnp.float32),
            pltpu.VMEM((_CHUNK * _B, 3 * _H), jnp.float32),
            pltpu.VMEM((_CHUNK * _B, 3 * _H), jnp.float32),
            pltpu.VMEM((_CHUNK * _B, _H), jnp.float32),
        ],
        compiler_params=pltpu.CompilerParams(
            dimension_semantics=("arbitrary",),
        ),
    )(
        xt,
        w_ih_l0.T.astype(jnp.bfloat16), w_hh_l0.T.astype(jnp.bfloat16),
        b_ih_l0[None], b_hh_l0[None],
        w_ih_l1.T.astype(jnp.bfloat16), w_hh_l1.T.astype(jnp.bfloat16),
        b_ih_l1[None], b_hh_l1[None],
    )
    return jnp.swapaxes(y.reshape(_T, _B, _H), 0, 1)


# speedup vs baseline: 1.7843x; 1.0185x over previous
import jax
import jax.numpy as jnp
from jax.experimental import pallas as pl
from jax.experimental.pallas import tpu as pltpu

_B, _T, _D, _H = 8, 2048, 256, 256
_CHUNK = 64
_NCH = _T // _CHUNK


def _gru2_kernel(x_ref, wih0_ref, whh0_ref, bih0_ref, bhh0_ref,
                 wih1_ref, whh1_ref, bih1_ref, bhh1_ref,
                 y_ref, h0_ref, h1_ref, gi0_ref, gi1_ref, y0_ref):
    c = pl.program_id(0)

    @pl.when(c == 0)
    def _init0():
        h0_ref[...] = jnp.zeros_like(h0_ref)

    @pl.when(c <= 1)
    def _init1():
        h1_ref[...] = jnp.zeros_like(h1_ref)

    gi0_ref[...] = (
        jnp.dot(x_ref[...], wih0_ref[...], preferred_element_type=jnp.float32)
        + bih0_ref[...]
    )

    def gates(g_i, g_h, h):
        r = jax.nn.sigmoid(g_i[:, :_H] + g_h[:, :_H])
        z = jax.nn.sigmoid(g_i[:, _H:2 * _H] + g_h[:, _H:2 * _H])
        n = jnp.tanh(g_i[:, 2 * _H:] + r * g_h[:, 2 * _H:])
        return (1.0 - z) * n + z * h

    def layer_step(h, w_ref, b_ref, gi):
        hb16 = h.astype(jnp.bfloat16)
        gh_r = jnp.dot(hb16, w_ref[:, :_H],
                       preferred_element_type=jnp.float32) + b_ref[:, :_H]
        gh_z = jnp.dot(hb16, w_ref[:, _H:2 * _H],
                       preferred_element_type=jnp.float32) + b_ref[:, _H:2 * _H]
        gh_n = jnp.dot(hb16, w_ref[:, 2 * _H:],
                       preferred_element_type=jnp.float32) + b_ref[:, 2 * _H:]
        r = jax.nn.sigmoid(gi[:, :_H] + gh_r)
        z = jax.nn.sigmoid(gi[:, _H:2 * _H] + gh_z)
        n = jnp.tanh(gi[:, 2 * _H:] + r * gh_n)
        return (1.0 - z) * n + z * h

    def body(i, carry):
        h0, h1 = carry
        h0_next = layer_step(h0, whh0_ref, bhh0_ref, gi0_ref[pl.ds(i * _B, _B)])
        h1_next = layer_step(h1, whh1_ref, bhh1_ref, gi1_ref[pl.ds(i * _B, _B)])
        y0_ref[pl.ds(i * _B, _B)] = h0_next
        y_ref[pl.ds(i * _B, _B)] = h1_next
        return h0_next, h1_next

    h0, h1 = jax.lax.fori_loop(0, _CHUNK, body, (h0_ref[...], h1_ref[...]),
                               unroll=64)
    h0_ref[...] = h0
    h1_ref[...] = h1

    gi1_ref[...] = (
        jnp.dot(y0_ref[...].astype(jnp.bfloat16), wih1_ref[...],
                preferred_element_type=jnp.float32)
        + bih1_ref[...]
    )


@jax.jit
def kernel(x, w_ih_l0, w_hh_l0, b_ih_l0, b_hh_l0,
           w_ih_l1, w_hh_l1, b_ih_l1, b_hh_l1):
    xt = jnp.swapaxes(x, 0, 1).reshape(_T * _B, _D).astype(jnp.bfloat16)

    full = lambda shape: pl.BlockSpec(shape, lambda c: (0,) * len(shape))
    y = pl.pallas_call(
        _gru2_kernel,
        grid=(_NCH + 1,),
        in_specs=[
            pl.BlockSpec((_CHUNK * _B, _D),
                         lambda c: (jnp.minimum(c, _NCH - 1), 0)),
            full((_D, 3 * _H)),
            full((_H, 3 * _H)),
            full((1, 3 * _H)),
            full((1, 3 * _H)),
            full((_H, 3 * _H)),
            full((_H, 3 * _H)),
            full((1, 3 * _H)),
            full((1, 3 * _H)),
        ],
        out_specs=pl.BlockSpec((_CHUNK * _B, _H),
                               lambda c: (jnp.maximum(c - 1, 0), 0)),
        out_shape=jax.ShapeDtypeStruct((_T * _B, _H), jnp.float32),
        scratch_shapes=[
            pltpu.VMEM((_B, _H), jnp.float32),
            pltpu.VMEM((_B, _H), jnp.float32),
            pltpu.VMEM((_CHUNK * _B, 3 * _H), jnp.float32),
            pltpu.VMEM((_CHUNK * _B, 3 * _H), jnp.float32),
            pltpu.VMEM((_CHUNK * _B, _H), jnp.float32),
        ],
        compiler_params=pltpu.CompilerParams(
            dimension_semantics=("arbitrary",),
        ),
    )(
        xt,
        w_ih_l0.T.astype(jnp.bfloat16), w_hh_l0.T.astype(jnp.bfloat16),
        b_ih_l0[None], b_hh_l0[None],
        w_ih_l1.T.astype(jnp.bfloat16), w_hh_l1.T.astype(jnp.bfloat16),
        b_ih_l1[None], b_hh_l1[None],
    )
    return jnp.swapaxes(y.reshape(_T, _B, _H), 0, 1)
